# baseline (speedup 1.0000x reference)
import functools

import jax
import jax.numpy as jnp
from jax import lax
from jax.experimental import pallas as pl
from jax.experimental.pallas import tpu as pltpu, tpu_sc as plsc

F_TOTAL = 4
N_ROWS = 100000
D = 64
B = 16384

_INFO = plsc.get_sparse_core_info()
NC, NS, L = _INFO.num_cores, _INFO.num_subcores, _INFO.num_lanes
NW = NC * NS
BPW = B // NW
CHUNK = 128
NCHUNK = BPW // CHUNK
GROUPS = CHUNK // 16


def _sc_body(feats_hbm, aux_hbm, idx_i_hbm, idx_j_hbm, out_hbm,
             idx_i_v, idx_j_v, idxflat_v, t0, t1, t2, t3, t4, t5, t6, t7,
             out_v, aux_v, sem):
    tmps = (t0, t1, t2, t3, t4, t5, t6, t7)
    wid = lax.axis_index("s") * NC + lax.axis_index("c")
    base = wid * BPW

    pltpu.sync_copy(idx_i_hbm.at[pl.ds(base, BPW)], idx_i_v)
    pltpu.sync_copy(idx_j_hbm.at[pl.ds(base, BPW)], idx_j_v)
    pltpu.sync_copy(aux_hbm, aux_v)

    lane = lax.iota(jnp.int32, L)
    aux = aux_v[...]
    fw = jnp.where(lane < F_TOTAL, aux, jnp.float32(-1e30))
    m = jnp.max(fw)
    e = jnp.exp(fw - m)
    w_vec = e / jnp.sum(e)
    w = [jnp.sum(jnp.where(lane == f, w_vec, jnp.float32(0.0)))
         for f in range(F_TOTAL)]
    intercept = jnp.sum(jnp.where(lane == F_TOTAL, aux, jnp.float32(0.0)))
    slope = jnp.sum(jnp.where(lane == F_TOTAL + 1, aux, jnp.float32(0.0)))

    for c16 in range(BPW // L):
        vi = idx_i_v[pl.ds(c16 * L, L)]
        vj = idx_j_v[pl.ds(c16 * L, L)]
        ch, off = (c16 * L) // CHUNK, (c16 * L) % CHUNK
        for f in range(F_TOTAL):
            idxflat_v[f, ch, pl.ds(off, L)] = vi + jnp.int32(f * N_ROWS)
            idxflat_v[F_TOTAL + f, ch, pl.ds(off, L)] = vj + jnp.int32(f * N_ROWS)

    for c in range(NCHUNK):
        copies = [pltpu.make_async_copy(feats_hbm.at[idxflat_v.at[k, c]],
                                        tmps[k], sem)
                  for k in range(8)]
        for cp in copies:
            cp.start()
        for cp in copies:
            cp.wait()

        def g_body(g, _):
            rows = lane + g * jnp.int32(L)

            def d_body(dd, s):
                cols = jnp.full((L,), dd, jnp.int32)
                acc = w[0] * plsc.load_gather(t0, [rows, cols])
                acc += w[1] * plsc.load_gather(t1, [rows, cols])
                acc += w[2] * plsc.load_gather(t2, [rows, cols])
                acc += w[3] * plsc.load_gather(t3, [rows, cols])
                acc -= w[0] * plsc.load_gather(t4, [rows, cols])
                acc -= w[1] * plsc.load_gather(t5, [rows, cols])
                acc -= w[2] * plsc.load_gather(t6, [rows, cols])
                acc -= w[3] * plsc.load_gather(t7, [rows, cols])
                return s + acc * acc

            dist = lax.fori_loop(0, D, d_body, jnp.zeros((L,), jnp.float32))
            logits = intercept - slope * dist
            out_v[pl.ds(c * CHUNK + g * L, L)] = (
                jnp.float32(1.0) / (jnp.float32(1.0) + jnp.exp(-logits)))
            return 0

        lax.fori_loop(0, GROUPS, g_body, 0)

    pltpu.sync_copy(out_v, out_hbm.at[pl.ds(base, BPW)])


@functools.partial(jax.jit, static_argnames=())
def kernel(features, feature_weights, intercept, slope, idx_i, idx_j):
    feats_flat = features.reshape(F_TOTAL * N_ROWS, D)
    aux = jnp.zeros((L,), jnp.float32)
    aux = aux.at[:F_TOTAL].set(feature_weights.reshape(-1).astype(jnp.float32))
    aux = aux.at[F_TOTAL].set(intercept.astype(jnp.float32))
    aux = aux.at[F_TOTAL + 1].set(slope.astype(jnp.float32))

    mesh = plsc.VectorSubcoreMesh(core_axis_name="c", subcore_axis_name="s")
    run = pl.kernel(
        _sc_body,
        mesh=mesh,
        out_type=jax.ShapeDtypeStruct((B,), jnp.float32),
        compiler_params=pltpu.CompilerParams(
            needs_layout_passes=False, use_tc_tiling_on_sc=False),
        scratch_types=[
            pltpu.VMEM((BPW,), jnp.int32),
            pltpu.VMEM((BPW,), jnp.int32),
            pltpu.VMEM((8, NCHUNK, CHUNK), jnp.int32),
        ] + [pltpu.VMEM((CHUNK, D), jnp.float32) for _ in range(8)]
        + [
            pltpu.VMEM((BPW,), jnp.float32),
            pltpu.VMEM((L,), jnp.float32),
            pltpu.SemaphoreType.DMA,
        ],
    )
    return run(feats_flat, aux, idx_i, idx_j)


# speedup vs baseline: 1.6180x; 1.6180x over previous
import functools

import jax
import jax.numpy as jnp
from jax import lax
from jax.experimental import pallas as pl
from jax.experimental.pallas import tpu as pltpu, tpu_sc as plsc

F_TOTAL = 4
N_ROWS = 100000
D = 64
B = 16384
EW = 128

_INFO = plsc.get_sparse_core_info()
NC, NS, L = _INFO.num_cores, _INFO.num_subcores, _INFO.num_lanes
NW = NC * NS
BPW = B // NW
CHUNK = 128
NCHUNK = BPW // CHUNK
GROUPS = CHUNK // 16

NB = 512
GRID = (N_ROWS + NB - 1) // NB


def _mix_body(aux_ref, feats_ref, out_ref):
    a = aux_ref[:, :F_TOTAL]
    e = jnp.exp(a - jnp.max(a))
    w = e / jnp.sum(e)
    x = feats_ref[...]
    mix = (w[0, 0] * x[0] + w[0, 1] * x[1] + w[0, 2] * x[2] + w[0, 3] * x[3])
    out_ref[:, :D] = mix.T


def _dist_body(e_hbm, aux_hbm, idx_i_hbm, idx_j_hbm, out_hbm,
               idxc_v, ti_v, tj_v, out_v, aux_v, sem):
    wid = lax.axis_index("s") * NC + lax.axis_index("c")
    base = wid * BPW

    pltpu.sync_copy(aux_hbm, aux_v)
    lane = lax.iota(jnp.int32, L)
    aux = aux_v[...]
    intercept = jnp.sum(jnp.where(lane == F_TOTAL, aux, jnp.float32(0.0)))
    slope = jnp.sum(jnp.where(lane == F_TOTAL + 1, aux, jnp.float32(0.0)))

    for c in range(NCHUNK):
        pltpu.sync_copy(idx_i_hbm.at[pl.ds(base + c * CHUNK, CHUNK)],
                        idxc_v.at[0, c])
        pltpu.sync_copy(idx_j_hbm.at[pl.ds(base + c * CHUNK, CHUNK)],
                        idxc_v.at[1, c])
        cpi = pltpu.make_async_copy(e_hbm.at[idxc_v.at[0, c]], ti_v, sem)
        cpj = pltpu.make_async_copy(e_hbm.at[idxc_v.at[1, c]], tj_v, sem)
        cpi.start()
        cpj.start()
        cpi.wait()
        cpj.wait()

        def g_body(g, _):
            rows = lane + g * jnp.int32(L)

            def d_body(dd, s):
                cols = jnp.full((L,), dd, jnp.int32)
                diff = (plsc.load_gather(ti_v, [rows, cols])
                        - plsc.load_gather(tj_v, [rows, cols]))
                return s + diff * diff

            dist = lax.fori_loop(0, D, d_body, jnp.zeros((L,), jnp.float32))
            logits = intercept - slope * dist
            out_v[pl.ds(c * CHUNK + g * L, L)] = (
                jnp.float32(1.0) / (jnp.float32(1.0) + jnp.exp(-logits)))
            return 0

        lax.fori_loop(0, GROUPS, g_body, 0)

    pltpu.sync_copy(out_v, out_hbm.at[pl.ds(base, BPW)])


@jax.jit
def kernel(features, feature_weights, intercept, slope, idx_i, idx_j):
    feats_t = features.transpose(0, 2, 1)
    aux = jnp.zeros((L,), jnp.float32)
    aux = aux.at[:F_TOTAL].set(feature_weights.reshape(-1).astype(jnp.float32))
    aux = aux.at[F_TOTAL].set(intercept.astype(jnp.float32))
    aux = aux.at[F_TOTAL + 1].set(slope.astype(jnp.float32))

    e_table = pl.pallas_call(
        _mix_body,
        grid=(GRID,),
        in_specs=[
            pl.BlockSpec((1, L), lambda i: (0, 0)),
            pl.BlockSpec((F_TOTAL, D, NB), lambda i: (0, 0, i)),
        ],
        out_specs=pl.BlockSpec((NB, EW), lambda i: (i, 0)),
        out_shape=jax.ShapeDtypeStruct((GRID * NB, EW), jnp.float32),
    )(aux.reshape(1, L), feats_t)

    mesh = plsc.VectorSubcoreMesh(core_axis_name="c", subcore_axis_name="s")
    run = pl.kernel(
        _dist_body,
        mesh=mesh,
        out_type=jax.ShapeDtypeStruct((B,), jnp.float32),
        compiler_params=pltpu.CompilerParams(
            needs_layout_passes=False, use_tc_tiling_on_sc=True),
        scratch_types=[
            pltpu.VMEM((2, NCHUNK, CHUNK), jnp.int32),
            pltpu.VMEM((CHUNK, EW), jnp.float32),
            pltpu.VMEM((CHUNK, EW), jnp.float32),
            pltpu.VMEM((BPW,), jnp.float32),
            pltpu.VMEM((L,), jnp.float32),
            pltpu.SemaphoreType.DMA,
        ],
    )
    return run(e_table, aux, idx_i, idx_j)


# speedup vs baseline: 2.7880x; 1.7231x over previous
import functools

import jax
import jax.numpy as jnp
from jax import lax
from jax.experimental import pallas as pl
from jax.experimental.pallas import tpu as pltpu, tpu_sc as plsc

F_TOTAL = 4
N_ROWS = 100000
D = 64
B = 16384
EW = 128

_INFO = plsc.get_sparse_core_info()
NC, NS, L = _INFO.num_cores, _INFO.num_subcores, _INFO.num_lanes
NW = NC * NS
BPW = B // NW
CHUNK = 128
NCHUNK = BPW // CHUNK
GROUPS = CHUNK // 16

NB = 2048
GRID = (N_ROWS + NB - 1) // NB


def _mix_body(aux_ref, feats_ref, out_ref):
    a = aux_ref[:, :F_TOTAL]
    e = jnp.exp(a - jnp.max(a))
    w = e / jnp.sum(e)
    x = feats_ref[...]
    mix = (w[0, 0] * x[0] + w[0, 1] * x[1] + w[0, 2] * x[2] + w[0, 3] * x[3])
    out_ref[:, :D] = mix.T


def _dist_body(e_hbm, aux_hbm, idx_i_hbm, idx_j_hbm, out_hbm,
               idxc_v, ti_v, tj_v, out_v, aux_v, sem):
    wid = lax.axis_index("s") * NC + lax.axis_index("c")
    base = wid * BPW

    pltpu.sync_copy(aux_hbm, aux_v)
    lane = lax.iota(jnp.int32, L)
    aux = aux_v[...]
    intercept = jnp.sum(jnp.where(lane == F_TOTAL, aux, jnp.float32(0.0)))
    slope = jnp.sum(jnp.where(lane == F_TOTAL + 1, aux, jnp.float32(0.0)))

    for c in range(NCHUNK):
        pltpu.sync_copy(idx_i_hbm.at[pl.ds(base + c * CHUNK, CHUNK)],
                        idxc_v.at[0, c])
        pltpu.sync_copy(idx_j_hbm.at[pl.ds(base + c * CHUNK, CHUNK)],
                        idxc_v.at[1, c])
        cpi = pltpu.make_async_copy(e_hbm.at[idxc_v.at[0, c]], ti_v, sem)
        cpj = pltpu.make_async_copy(e_hbm.at[idxc_v.at[1, c]], tj_v, sem)
        cpi.start()
        cpj.start()
        cpi.wait()
        cpj.wait()

        def g_body(g, _):
            rows = lane + g * jnp.int32(L)

            def d_body(dd, s):
                cols = jnp.full((L,), dd, jnp.int32)
                diff = (plsc.load_gather(ti_v, [rows, cols])
                        - plsc.load_gather(tj_v, [rows, cols]))
                return s + diff * diff

            dist = lax.fori_loop(0, D, d_body, jnp.zeros((L,), jnp.float32))
            logits = intercept - slope * dist
            out_v[pl.ds(c * CHUNK + g * L, L)] = (
                jnp.float32(1.0) / (jnp.float32(1.0) + jnp.exp(-logits)))
            return 0

        lax.fori_loop(0, GROUPS, g_body, 0)

    pltpu.sync_copy(out_v, out_hbm.at[pl.ds(base, BPW)])


@jax.jit
def kernel(features, feature_weights, intercept, slope, idx_i, idx_j):
    feats_t = features.transpose(0, 2, 1)
    aux = jnp.zeros((L,), jnp.float32)
    aux = aux.at[:F_TOTAL].set(feature_weights.reshape(-1).astype(jnp.float32))
    aux = aux.at[F_TOTAL].set(intercept.astype(jnp.float32))
    aux = aux.at[F_TOTAL + 1].set(slope.astype(jnp.float32))

    e_table = pl.pallas_call(
        _mix_body,
        grid=(GRID,),
        in_specs=[
            pl.BlockSpec((1, L), lambda i: (0, 0)),
            pl.BlockSpec((F_TOTAL, D, NB), lambda i: (0, 0, i)),
        ],
        out_specs=pl.BlockSpec((NB, EW), lambda i: (i, 0)),
        out_shape=jax.ShapeDtypeStruct((GRID * NB, EW), jnp.float32),
    )(aux.reshape(1, L), feats_t)

    mesh = plsc.VectorSubcoreMesh(core_axis_name="c", subcore_axis_name="s")
    run = pl.kernel(
        _dist_body,
        mesh=mesh,
        out_type=jax.ShapeDtypeStruct((B,), jnp.float32),
        compiler_params=pltpu.CompilerParams(
            needs_layout_passes=False, use_tc_tiling_on_sc=True),
        scratch_types=[
            pltpu.VMEM((2, NCHUNK, CHUNK), jnp.int32),
            pltpu.VMEM((CHUNK, EW), jnp.float32),
            pltpu.VMEM((CHUNK, EW), jnp.float32),
            pltpu.VMEM((BPW,), jnp.float32),
            pltpu.VMEM((L,), jnp.float32),
            pltpu.SemaphoreType.DMA,
        ],
    )
    return run(e_table, aux, idx_i, idx_j)


# speedup vs baseline: 3.2051x; 1.1496x over previous
import functools

import jax
import jax.numpy as jnp
from jax import lax
from jax.experimental import pallas as pl
from jax.experimental.pallas import tpu as pltpu, tpu_sc as plsc

F_TOTAL = 4
N_ROWS = 100000
D = 64
B = 16384
EW = 128

_INFO = plsc.get_sparse_core_info()
NC, NS, L = _INFO.num_cores, _INFO.num_subcores, _INFO.num_lanes
NW = NC * NS
BPW = B // NW
CHUNK = 128
NCHUNK = BPW // CHUNK
GROUPS = CHUNK // 16

NB = 4096
GRID = (N_ROWS + NB - 1) // NB


def _mix_body(aux_ref, feats_ref, out_ref):
    a = aux_ref[:, :F_TOTAL]
    e = jnp.exp(a - jnp.max(a))
    w = e / jnp.sum(e)
    x = feats_ref[...]
    mix = (w[0, 0] * x[0] + w[0, 1] * x[1] + w[0, 2] * x[2] + w[0, 3] * x[3])
    out_ref[:, :D] = mix.T


def _dist_body(e_hbm, aux_hbm, idx_i_hbm, idx_j_hbm, out_hbm,
               idxc_v, ti_v, tj_v, out_v, aux_v, sem):
    wid = lax.axis_index("s") * NC + lax.axis_index("c")
    base = wid * BPW

    pltpu.sync_copy(aux_hbm, aux_v)
    lane = lax.iota(jnp.int32, L)
    aux = aux_v[...]
    intercept = jnp.sum(jnp.where(lane == F_TOTAL, aux, jnp.float32(0.0)))
    slope = jnp.sum(jnp.where(lane == F_TOTAL + 1, aux, jnp.float32(0.0)))

    for c in range(NCHUNK):
        pltpu.sync_copy(idx_i_hbm.at[pl.ds(base + c * CHUNK, CHUNK)],
                        idxc_v.at[0, c])
        pltpu.sync_copy(idx_j_hbm.at[pl.ds(base + c * CHUNK, CHUNK)],
                        idxc_v.at[1, c])
        cpi = pltpu.make_async_copy(e_hbm.at[idxc_v.at[0, c]], ti_v, sem)
        cpj = pltpu.make_async_copy(e_hbm.at[idxc_v.at[1, c]], tj_v, sem)
        cpi.start()
        cpj.start()
        cpi.wait()
        cpj.wait()

        def g_body(g, _):
            rows = lane + g * jnp.int32(L)

            def d_body(dd, s):
                cols = jnp.full((L,), dd, jnp.int32)
                diff = (plsc.load_gather(ti_v, [rows, cols])
                        - plsc.load_gather(tj_v, [rows, cols]))
                return s + diff * diff

            dist = lax.fori_loop(0, D, d_body, jnp.zeros((L,), jnp.float32))
            logits = intercept - slope * dist
            out_v[pl.ds(c * CHUNK + g * L, L)] = (
                jnp.float32(1.0) / (jnp.float32(1.0) + jnp.exp(-logits)))
            return 0

        lax.fori_loop(0, GROUPS, g_body, 0)

    pltpu.sync_copy(out_v, out_hbm.at[pl.ds(base, BPW)])


@jax.jit
def kernel(features, feature_weights, intercept, slope, idx_i, idx_j):
    feats_t = features.transpose(0, 2, 1)
    aux = jnp.zeros((L,), jnp.float32)
    aux = aux.at[:F_TOTAL].set(feature_weights.reshape(-1).astype(jnp.float32))
    aux = aux.at[F_TOTAL].set(intercept.astype(jnp.float32))
    aux = aux.at[F_TOTAL + 1].set(slope.astype(jnp.float32))

    e_table = pl.pallas_call(
        _mix_body,
        grid=(GRID,),
        in_specs=[
            pl.BlockSpec((1, L), lambda i: (0, 0)),
            pl.BlockSpec((F_TOTAL, D, NB), lambda i: (0, 0, i)),
        ],
        out_specs=pl.BlockSpec((NB, EW), lambda i: (i, 0)),
        out_shape=jax.ShapeDtypeStruct((GRID * NB, EW), jnp.float32),
    )(aux.reshape(1, L), feats_t)

    mesh = plsc.VectorSubcoreMesh(core_axis_name="c", subcore_axis_name="s")
    run = pl.kernel(
        _dist_body,
        mesh=mesh,
        out_type=jax.ShapeDtypeStruct((B,), jnp.float32),
        compiler_params=pltpu.CompilerParams(
            needs_layout_passes=False, use_tc_tiling_on_sc=True),
        scratch_types=[
            pltpu.VMEM((2, NCHUNK, CHUNK), jnp.int32),
            pltpu.VMEM((CHUNK, EW), jnp.float32),
            pltpu.VMEM((CHUNK, EW), jnp.float32),
            pltpu.VMEM((BPW,), jnp.float32),
            pltpu.VMEM((L,), jnp.float32),
            pltpu.SemaphoreType.DMA,
        ],
    )
    return run(e_table, aux, idx_i, idx_j)
